# baseline (speedup 1.0000x reference)
import functools

import jax
import jax.numpy as jnp
from jax import lax
from jax.experimental import pallas as pl
from jax.experimental.pallas import tpu as pltpu
from jax.experimental.pallas import tpu_sc as plsc

VOCAB = 8192
DIM = 64
PAIRS = 48
HW = 128 * 128
NW = 32
ROWS_PER_W = 8
SHARERS = NW * ROWS_PER_W // DIM
CHUNK = 2048
NCHUNK = HW // CHUNK
GROUPS = CHUNK // 16
NBUF = 2
NUNITS = PAIRS * NCHUNK


def _transpose_body(t_ref, o_ref):
    o_ref[...] = t_ref[...].T.reshape(8, 8, 128)


def _transpose_table(table):
    return pl.pallas_call(
        _transpose_body,
        grid=(VOCAB // 128,),
        in_specs=[pl.BlockSpec((128, DIM), lambda i: (i, 0))],
        out_specs=pl.BlockSpec((8, 8, 128), lambda i: (0, i, 0)),
        out_shape=jax.ShapeDtypeStruct((8, (VOCAB // 128) * 8, 128), jnp.float32),
    )(table)


_MESH = plsc.VectorSubcoreMesh(core_axis_name="c", subcore_axis_name="s")


@functools.partial(
    pl.kernel,
    out_type=jax.ShapeDtypeStruct((PAIRS * DIM * HW,), jnp.float32),
    mesh=_MESH,
    compiler_params=pltpu.CompilerParams(needs_layout_passes=False),
    scratch_types=[
        pltpu.VMEM(((VOCAB // 128) * 8, 128), jnp.float32),
        pltpu.VMEM((NBUF * CHUNK,), jnp.int32),
        pltpu.VMEM((NBUF * ROWS_PER_W * CHUNK,), jnp.float32),
        pltpu.SemaphoreType.DMA,
        pltpu.SemaphoreType.DMA,
        pltpu.SemaphoreType.DMA,
        pltpu.SemaphoreType.DMA,
    ],
)
def _sc_gather(tabT_hbm, idx_hbm, out_hbm, tab_v, idx_v, out_v,
               in_sem0, in_sem1, out_sem0, out_sem1):
    in_sems = (in_sem0, in_sem1)
    out_sems = (out_sem0, out_sem1)
    wid = lax.axis_index("s") * 2 + lax.axis_index("c")
    d0 = (wid // SHARERS) * ROWS_PER_W
    ubase = (wid % SHARERS) * (NUNITS // SHARERS)
    pltpu.sync_copy(tabT_hbm.at[d0 // 8], tab_v)

    def idx_copy(b, u):
        return pltpu.make_async_copy(
            idx_hbm.at[pl.ds(u * CHUNK, CHUNK)],
            idx_v.at[pl.ds(b * CHUNK, CHUNK)],
            in_sems[b],
        )

    def out_copy(b, j, r, col):
        return pltpu.make_async_copy(
            out_v.at[pl.ds((b * ROWS_PER_W + j) * CHUNK, CHUNK)],
            out_hbm.at[pl.ds(r * HW + col, CHUNK)],
            out_sems[b],
        )

    for b in range(NBUF):
        idx_copy(b, ubase + b).start()

    @pl.loop(0, NUNITS // SHARERS, step=NBUF)
    def unit_loop(u0):
        for b in range(NBUF):
            u = ubase + u0 + b
            idx_copy(b, u).wait()

            @pl.when(u0 >= NBUF)
            def _():
                for j in range(ROWS_PER_W):
                    out_copy(b, j, 0, 0).wait()

            zero16 = jnp.zeros((16,), jnp.int32)

            @plsc.parallel_loop(0, GROUPS, unroll=8)
            def gather_loop(g):
                ids = idx_v[pl.ds(b * CHUNK + g * 16, 16)]
                base = ids + (ids // 128) * ((8 - 1) * 128)
                for j in range(ROWS_PER_W):
                    out_v[pl.ds((b * ROWS_PER_W + j) * CHUNK + g * 16, 16)] = (
                        plsc.load_gather(tab_v, [zero16, base + j * 128])
                    )

            @pl.when(u0 + b + NBUF < NUNITS // SHARERS)
            def _():
                idx_copy(b, u + NBUF).start()

            pair = u // NCHUNK
            chunk = u % NCHUNK
            r = pair * DIM + d0
            col = chunk * CHUNK
            for j in range(ROWS_PER_W):
                out_copy(b, j, r + j, col).start()

    for b in range(NBUF):
        for j in range(ROWS_PER_W):
            out_copy(b, j, 0, 0).wait()


def kernel(images, table):
    b, c, h, w = images.shape
    tabT = _transpose_table(table)
    idx = images.astype(jnp.int32).reshape(PAIRS * HW)
    out = _sc_gather(tabT, idx)
    return out.reshape(b, c * DIM, h, w)


# speedup vs baseline: 1.2127x; 1.2127x over previous
import functools

import jax
import jax.numpy as jnp
from jax import lax
from jax.experimental import pallas as pl
from jax.experimental.pallas import tpu as pltpu
from jax.experimental.pallas import tpu_sc as plsc

VOCAB = 8192
DIM = 64
PAIRS = 48
HW = 128 * 128
NW = 32
ROWS_PER_W = 8
SHARERS = NW * ROWS_PER_W // DIM
CHUNK = 2048
NCHUNK = HW // CHUNK
GROUPS = CHUNK // 16
NBUF = 2
NUNITS = PAIRS * NCHUNK


_TSTEPS = 4
_TROWS = VOCAB // _TSTEPS


def _transpose_body(t_ref, o_ref):
    for k in range(_TROWS // 128):
        x = t_ref[pl.ds(k * 128, 128), :]
        o_ref[:, pl.ds(k * 8, 8), :] = x.T.reshape(8, 8, 128)


def _transpose_table(table):
    return pl.pallas_call(
        _transpose_body,
        grid=(_TSTEPS,),
        in_specs=[pl.BlockSpec((_TROWS, DIM), lambda i: (i, 0))],
        out_specs=pl.BlockSpec((8, _TROWS // 16, 128), lambda i: (0, i, 0)),
        out_shape=jax.ShapeDtypeStruct((8, (VOCAB // 128) * 8, 128), jnp.float32),
    )(table)


_MESH = plsc.VectorSubcoreMesh(core_axis_name="c", subcore_axis_name="s")


@functools.partial(
    pl.kernel,
    out_type=jax.ShapeDtypeStruct((PAIRS * DIM * HW,), jnp.float32),
    mesh=_MESH,
    compiler_params=pltpu.CompilerParams(needs_layout_passes=False),
    scratch_types=[
        pltpu.VMEM(((VOCAB // 128) * 8, 128), jnp.float32),
        pltpu.VMEM((NBUF * CHUNK,), jnp.int32),
        pltpu.VMEM((NBUF * ROWS_PER_W * CHUNK,), jnp.float32),
        pltpu.SemaphoreType.DMA,
        pltpu.SemaphoreType.DMA,
        pltpu.SemaphoreType.DMA,
        pltpu.SemaphoreType.DMA,
    ],
)
def _sc_gather(tabT_hbm, idx_hbm, out_hbm, tab_v, idx_v, out_v,
               in_sem0, in_sem1, out_sem0, out_sem1):
    in_sems = (in_sem0, in_sem1)
    out_sems = (out_sem0, out_sem1)
    wid = lax.axis_index("s") * 2 + lax.axis_index("c")
    d0 = (wid // SHARERS) * ROWS_PER_W
    ubase = (wid % SHARERS) * (NUNITS // SHARERS)
    pltpu.sync_copy(tabT_hbm.at[d0 // 8], tab_v)

    def idx_copy(b, u):
        return pltpu.make_async_copy(
            idx_hbm.at[pl.ds(u * CHUNK, CHUNK)],
            idx_v.at[pl.ds(b * CHUNK, CHUNK)],
            in_sems[b],
        )

    def out_copy(b, j, r, col):
        return pltpu.make_async_copy(
            out_v.at[pl.ds((b * ROWS_PER_W + j) * CHUNK, CHUNK)],
            out_hbm.at[pl.ds(r * HW + col, CHUNK)],
            out_sems[b],
        )

    for b in range(NBUF):
        idx_copy(b, ubase + b).start()

    @pl.loop(0, NUNITS // SHARERS, step=NBUF)
    def unit_loop(u0):
        for b in range(NBUF):
            u = ubase + u0 + b
            idx_copy(b, u).wait()

            @pl.when(u0 >= NBUF)
            def _():
                for j in range(ROWS_PER_W):
                    out_copy(b, j, 0, 0).wait()

            zero16 = jnp.zeros((16,), jnp.int32)

            @plsc.parallel_loop(0, GROUPS, unroll=8)
            def gather_loop(g):
                ids = idx_v[pl.ds(b * CHUNK + g * 16, 16)]
                base = ids + (ids // 128) * ((8 - 1) * 128)
                for j in range(ROWS_PER_W):
                    out_v[pl.ds((b * ROWS_PER_W + j) * CHUNK + g * 16, 16)] = (
                        plsc.load_gather(tab_v, [zero16, base + j * 128])
                    )

            @pl.when(u0 + b + NBUF < NUNITS // SHARERS)
            def _():
                idx_copy(b, u + NBUF).start()

            pair = u // NCHUNK
            chunk = u % NCHUNK
            r = pair * DIM + d0
            col = chunk * CHUNK
            for j in range(ROWS_PER_W):
                out_copy(b, j, r + j, col).start()

    for b in range(NBUF):
        for j in range(ROWS_PER_W):
            out_copy(b, j, 0, 0).wait()


def kernel(images, table):
    b, c, h, w = images.shape
    tabT = _transpose_table(table)
    idx = images.astype(jnp.int32).reshape(PAIRS * HW)
    out = _sc_gather(tabT, idx)
    return out.reshape(b, c * DIM, h, w)


# speedup vs baseline: 1.2210x; 1.0069x over previous
import functools

import jax
import jax.numpy as jnp
from jax import lax
from jax.experimental import pallas as pl
from jax.experimental.pallas import tpu as pltpu
from jax.experimental.pallas import tpu_sc as plsc

VOCAB = 8192
DIM = 64
PAIRS = 48
HW = 128 * 128
NW = 32
ROWS_PER_W = 8
SHARERS = NW * ROWS_PER_W // DIM
CHUNK = 2048
NCHUNK = HW // CHUNK
GROUPS = CHUNK // 16
NBUF = 2
NUNITS = PAIRS * NCHUNK


_TSTEPS = 2
_TROWS = VOCAB // _TSTEPS


def _transpose_body(t_ref, o_ref):
    for k in range(_TROWS // 128):
        x = t_ref[pl.ds(k * 128, 128), :]
        o_ref[:, pl.ds(k * 8, 8), :] = x.T.reshape(8, 8, 128)


def _transpose_table(table):
    return pl.pallas_call(
        _transpose_body,
        grid=(_TSTEPS,),
        in_specs=[pl.BlockSpec((_TROWS, DIM), lambda i: (i, 0))],
        out_specs=pl.BlockSpec((8, _TROWS // 16, 128), lambda i: (0, i, 0)),
        out_shape=jax.ShapeDtypeStruct((8, (VOCAB // 128) * 8, 128), jnp.float32),
    )(table)


_MESH = plsc.VectorSubcoreMesh(core_axis_name="c", subcore_axis_name="s")


@functools.partial(
    pl.kernel,
    out_type=jax.ShapeDtypeStruct((PAIRS * DIM * HW,), jnp.float32),
    mesh=_MESH,
    compiler_params=pltpu.CompilerParams(needs_layout_passes=False),
    scratch_types=[
        pltpu.VMEM(((VOCAB // 128) * 8, 128), jnp.float32),
        pltpu.VMEM((NBUF * CHUNK,), jnp.int32),
        pltpu.VMEM((NBUF * ROWS_PER_W * CHUNK,), jnp.float32),
        pltpu.SemaphoreType.DMA,
        pltpu.SemaphoreType.DMA,
        pltpu.SemaphoreType.DMA,
        pltpu.SemaphoreType.DMA,
    ],
)
def _sc_gather(tabT_hbm, idx_hbm, out_hbm, tab_v, idx_v, out_v,
               in_sem0, in_sem1, out_sem0, out_sem1):
    in_sems = (in_sem0, in_sem1)
    out_sems = (out_sem0, out_sem1)
    wid = lax.axis_index("s") * 2 + lax.axis_index("c")
    d0 = (wid // SHARERS) * ROWS_PER_W
    ubase = (wid % SHARERS) * (NUNITS // SHARERS)
    pltpu.sync_copy(tabT_hbm.at[d0 // 8], tab_v)

    def idx_copy(b, u):
        return pltpu.make_async_copy(
            idx_hbm.at[pl.ds(u * CHUNK, CHUNK)],
            idx_v.at[pl.ds(b * CHUNK, CHUNK)],
            in_sems[b],
        )

    def out_copy(b, j, r, col):
        return pltpu.make_async_copy(
            out_v.at[pl.ds((b * ROWS_PER_W + j) * CHUNK, CHUNK)],
            out_hbm.at[pl.ds(r * HW + col, CHUNK)],
            out_sems[b],
        )

    for b in range(NBUF):
        idx_copy(b, ubase + b).start()

    @pl.loop(0, NUNITS // SHARERS, step=NBUF)
    def unit_loop(u0):
        for b in range(NBUF):
            u = ubase + u0 + b
            idx_copy(b, u).wait()

            @pl.when(u0 >= NBUF)
            def _():
                for j in range(ROWS_PER_W):
                    out_copy(b, j, 0, 0).wait()

            zero16 = jnp.zeros((16,), jnp.int32)

            @plsc.parallel_loop(0, GROUPS, unroll=8)
            def gather_loop(g):
                ids = idx_v[pl.ds(b * CHUNK + g * 16, 16)]
                base = ids + (ids // 128) * ((8 - 1) * 128)
                for j in range(ROWS_PER_W):
                    out_v[pl.ds((b * ROWS_PER_W + j) * CHUNK + g * 16, 16)] = (
                        plsc.load_gather(tab_v, [zero16, base + j * 128])
                    )

            @pl.when(u0 + b + NBUF < NUNITS // SHARERS)
            def _():
                idx_copy(b, u + NBUF).start()

            pair = u // NCHUNK
            chunk = u % NCHUNK
            r = pair * DIM + d0
            col = chunk * CHUNK
            for j in range(ROWS_PER_W):
                out_copy(b, j, r + j, col).start()

    for b in range(NBUF):
        for j in range(ROWS_PER_W):
            out_copy(b, j, 0, 0).wait()


def kernel(images, table):
    b, c, h, w = images.shape
    tabT = _transpose_table(table)
    idx = images.astype(jnp.int32).reshape(PAIRS * HW)
    out = _sc_gather(tabT, idx)
    return out.reshape(b, c * DIM, h, w)
